# Initial kernel scaffold; baseline (speedup 1.0000x reference)
import jax
import jax.numpy as jnp
from jax.experimental import pallas as pl


def kernel(edge_index, edge_attr, bn_gamma, bn_beta, W_init, b_init, W_e1, b_e1, W_e2, b_e2, W_u, b_u, W_out, b_out):
    raise NotImplementedError("write your pallas kernel here")


# speedup vs baseline: 3.1567x; 3.1567x over previous
import functools

import jax
import jax.numpy as jnp
from jax import lax
from jax.experimental import pallas as pl
from jax.experimental.pallas import tpu as pltpu
from jax.experimental.pallas import tpu_sc as plsc

N = 10000
E = 320000
ED = 16
H = 128
OUT = 128
NUM_LAYERS = 3
EPS = 1e-5

NC = 2
NS = 16
NW = NC * NS
ROWS = E // 128
RPW = ROWS // NW
TAIL = ROWS - RPW * NW
STR = 624

_f32 = jnp.float32


def _mesh():
    return plsc.VectorSubcoreMesh(core_axis_name="c", subcore_axis_name="s")


def _striped_copy(s, src, dst):
    pltpu.sync_copy(src.at[pl.ds(s * STR, STR)], dst.at[pl.ds(s * STR, STR)])

    @pl.when(s == NS - 1)
    def _():
        rem = N - NS * STR
        pltpu.sync_copy(src.at[pl.ds(NS * STR, rem)], dst.at[pl.ds(NS * STR, rem)])


def _stats_body(ea_ref, sum_ref, sq_ref):
    i = pl.program_id(0)
    x = ea_ref[...]

    @pl.when(i == 0)
    def _():
        sum_ref[...] = jnp.zeros_like(sum_ref)
        sq_ref[...] = jnp.zeros_like(sq_ref)

    sum_ref[...] += jnp.sum(x, axis=0, keepdims=True)
    sq_ref[...] += jnp.sum(x * x, axis=0, keepdims=True)


def _stats(edge_attr):
    bs = 8000
    return pl.pallas_call(
        _stats_body,
        grid=(E // bs,),
        in_specs=[pl.BlockSpec((bs, ED), lambda i: (i, 0))],
        out_specs=[pl.BlockSpec((1, ED), lambda i: (0, 0))] * 2,
        out_shape=[jax.ShapeDtypeStruct((1, ED), _f32)] * 2,
    )(edge_attr)


KG = 6


def _gather_body(p_hbm, src3_hbm, g_hbm, gbuf, ibuf, sem):
    c = lax.axis_index("c")
    s = lax.axis_index("s")
    w = c * NS + s
    base = w * RPW

    @pl.loop(0, RPW // KG)
    def _chunk(k):
        r0 = base + k * KG
        pltpu.sync_copy(src3_hbm.at[pl.ds(r0, KG)], ibuf)
        descs = [
            pltpu.async_copy(p_hbm.at[ibuf.at[j, 0]], gbuf.at[pl.ds(j * 128, 128)], sem)
            for j in range(KG)
        ]
        for d in descs:
            d.wait()
        pltpu.sync_copy(gbuf, g_hbm.at[pl.ds(r0 * 128, KG * 128)])

    @pl.when(w < TAIL)
    def _tail():
        r = NW * RPW + w
        pltpu.sync_copy(src3_hbm.at[pl.ds(r, 1)], ibuf.at[pl.ds(0, 1)])
        pltpu.async_copy(p_hbm.at[ibuf.at[0, 0]], gbuf.at[pl.ds(0, 128)], sem).wait()
        pltpu.sync_copy(gbuf.at[pl.ds(0, 128)], g_hbm.at[pl.ds(r * 128, 128)])


def _gather(p, src3):
    fn = pl.kernel(
        _gather_body,
        out_type=jax.ShapeDtypeStruct((E, H), _f32),
        mesh=_mesh(),
        scratch_types=[
            pltpu.VMEM((KG * 128, H), _f32),
            pltpu.VMEM((KG, 1, 128), jnp.int32),
            pltpu.SemaphoreType.DMA,
        ],
    )
    return fn(p, src3)


KC = 3


def _scatter_body(msg_hbm, dst3_hbm, zero128_hbm, agg_hbm, mbuf, ibuf, acc):
    c = lax.axis_index("c")
    s = lax.axis_index("s")
    w = c * NS + s
    _striped_copy(s, zero128_hbm, acc)
    plsc.subcore_barrier()
    base = w * RPW

    @pl.loop(0, RPW // KC)
    def _chunk(k):
        r0 = base + k * KC
        pltpu.sync_copy(dst3_hbm.at[pl.ds(r0, KC)], ibuf)
        pltpu.sync_copy(msg_hbm.at[pl.ds(r0 * 128, KC * 128)], mbuf)
        for j in range(KC):
            pltpu.sync_copy(mbuf.at[pl.ds(j * 128, 128)], acc.at[ibuf.at[j, 0]], add=True)

    @pl.when(w < TAIL)
    def _tail():
        r = NW * RPW + w
        pltpu.sync_copy(dst3_hbm.at[pl.ds(r, 1)], ibuf.at[pl.ds(0, 1)])
        pltpu.sync_copy(msg_hbm.at[pl.ds(r * 128, 128)], mbuf.at[pl.ds(0, 128)])
        pltpu.sync_copy(mbuf.at[pl.ds(0, 128)], acc.at[ibuf.at[0, 0]], add=True)

    plsc.subcore_barrier()
    _striped_copy(s, acc, agg_hbm.at[c])


def _scatter(msg, dst3, zeros128):
    fn = pl.kernel(
        _scatter_body,
        out_type=jax.ShapeDtypeStruct((NC, N, H), _f32),
        mesh=_mesh(),
        scratch_types=[
            pltpu.VMEM((KC * 128, H), _f32),
            pltpu.VMEM((KC, 1, 128), jnp.int32),
            pltpu.VMEM_SHARED((N, H), _f32),
        ],
    )
    return fn(msg, dst3, zeros128)


CR = 8
CW = 1280
HN = CR * CW // 2
IPW = RPW * 128


def _counts_body(dst_hbm, cnt_hbm, ibufall, cbuf, obuf):
    c = lax.axis_index("c")
    s = lax.axis_index("s")
    w = c * NS + s
    lane = lax.iota(jnp.int32, 16)
    ones = jnp.ones((16,), jnp.int32)
    zeros = jnp.zeros((16,), jnp.int32)

    pltpu.sync_copy(dst_hbm.at[pl.ds(w * IPW, IPW)], ibufall.at[pl.ds(0, IPW)])

    @pl.when(w < TAIL)
    def _():
        pltpu.sync_copy(dst_hbm.at[pl.ds(NW * IPW + w * 128, 128)],
                        ibufall.at[pl.ds(IPW, 128)])

    nvec = jnp.where(w < TAIL, (IPW + 128) // 16, IPW // 16)

    for half in range(2):
        lo = half * HN

        @pl.loop(0, HN)
        def _zero(i):
            cbuf[pl.ds(i * 16, 16)] = zeros

        @pl.loop(0, nvec)
        def _count(i):
            idx = ibufall[pl.ds(i * 16, 16)]
            rel = idx - lo
            m = (rel >= 0) & (rel < HN)
            addr = rel * 16 + lane
            plsc.addupdate_scatter(cbuf, [addr], ones, mask=m)

        @pl.loop(0, HN // 16)
        def _reduce(gi):
            n0 = gi * 16
            base = n0 * 16 + lane * 16
            acc = plsc.load_gather(cbuf, [base])
            for l in range(1, 16):
                acc = acc + plsc.load_gather(cbuf, [base + l])
            flat = lo + n0
            obuf[flat // CW, pl.ds(flat % CW, 16)] = acc

    pltpu.sync_copy(obuf, cnt_hbm.at[w])


def _counts(dst1d):
    fn = pl.kernel(
        _counts_body,
        out_type=jax.ShapeDtypeStruct((NW, CR, CW), jnp.int32),
        mesh=_mesh(),
        scratch_types=[
            pltpu.VMEM((IPW + 128,), jnp.int32),
            pltpu.VMEM((HN * 16,), jnp.int32),
            pltpu.VMEM((CR, CW), jnp.int32),
        ],
        compiler_params=pltpu.CompilerParams(needs_layout_passes=False),
    )
    return fn(dst1d)


def _qinit_body(ea_ref, Wi_ref, q_ref):
    q_ref[...] = jnp.dot(ea_ref[...], Wi_ref[...], preferred_element_type=_f32)


def _qinit(edge_attr, Wi):
    return pl.pallas_call(
        _qinit_body,
        grid=(E // BM,),
        in_specs=[
            pl.BlockSpec((BM, ED), lambda i: (i, 0)),
            pl.BlockSpec((ED, H), lambda i: (0, 0)),
        ],
        out_specs=pl.BlockSpec((BM, H), lambda i: (i, 0)),
        out_shape=jax.ShapeDtypeStruct((E, H), _f32),
    )(edge_attr, Wi)


def _prep_body(qagg_ref, cnt_ref, bit_ref, bi_ref, W1b_ref, p_ref, cinv_ref):
    cnt = cnt_ref[...]
    rin = 1.0 / jnp.maximum(cnt, 1.0)
    nz = (cnt > 0.0).astype(_f32)
    x0 = jax.nn.relu(
        (qagg_ref[0] + qagg_ref[1]) * rin + nz * bit_ref[...] + bi_ref[...])
    p_ref[...] = jnp.dot(x0, W1b_ref[...], preferred_element_type=_f32)
    cinv_ref[...] = rin


def _prep(qagg, cnt, bit, bi, W1b):
    return pl.pallas_call(
        _prep_body,
        out_shape=[
            jax.ShapeDtypeStruct((N, H), _f32),
            jax.ShapeDtypeStruct((N, 1), _f32),
        ],
    )(qagg, cnt, bit, bi, W1b)


BM = 4000


def _msg_body(ea_ref, g_ref, W1_ref, b1_ref, W2_ref, b2_ref, out_ref):
    h = jax.nn.relu(
        jnp.dot(ea_ref[...], W1_ref[...], preferred_element_type=_f32)
        + g_ref[...] + b1_ref[...])
    out_ref[...] = jax.nn.relu(
        jnp.dot(h, W2_ref[...], preferred_element_type=_f32) + b2_ref[...])


def _msg(edge_attr, g, W1ap, b1p, We2, be2):
    return pl.pallas_call(
        _msg_body,
        grid=(E // BM,),
        in_specs=[
            pl.BlockSpec((BM, ED), lambda i: (i, 0)),
            pl.BlockSpec((BM, H), lambda i: (i, 0)),
            pl.BlockSpec((ED, H), lambda i: (0, 0)),
            pl.BlockSpec((1, H), lambda i: (0, 0)),
            pl.BlockSpec((H, H), lambda i: (0, 0)),
            pl.BlockSpec((1, H), lambda i: (0, 0)),
        ],
        out_specs=pl.BlockSpec((BM, H), lambda i: (i, 0)),
        out_shape=jax.ShapeDtypeStruct((E, H), _f32),
    )(edge_attr, g, W1ap, b1p, We2, be2)


def _update_body(aggp_ref, cinv_ref, Wu_ref, bu_ref, Wn_ref, bn_ref, out_ref):
    agg = (aggp_ref[0] + aggp_ref[1]) * cinv_ref[...]
    x = jax.nn.relu(
        jnp.dot(agg, Wu_ref[...], preferred_element_type=_f32) + bu_ref[...])
    out_ref[...] = jnp.dot(x, Wn_ref[...], preferred_element_type=_f32) + bn_ref[...]


def _update(aggp, cinv, Wu, bu, Wn, bn):
    return pl.pallas_call(
        _update_body,
        out_shape=jax.ShapeDtypeStruct((N, Wn.shape[1]), _f32),
    )(aggp, cinv, Wu, bu, Wn, bn)


def kernel(edge_index, edge_attr, bn_gamma, bn_beta, W_init, b_init,
           W_e1, b_e1, W_e2, b_e2, W_u, b_u, W_out, b_out):
    src3 = edge_index[0].reshape(ROWS, 1, 128)
    dst3 = edge_index[1].reshape(ROWS, 1, 128)

    ssum, ssq = _stats(edge_attr)
    mu = ssum / float(E)
    var = ssq / float(E) - mu * mu
    sv = bn_gamma[None, :] * lax.rsqrt(var + EPS)
    tv = bn_beta[None, :] - mu * sv
    W1a = W_e1[:ED]
    W1b = W_e1[ED:]
    W1ap = W1a * sv.reshape(ED, 1)
    b1p = tv @ W1a + b_e1[None, :]
    Wi = W_init * sv.reshape(ED, 1)
    bit = tv @ W_init

    zeros128 = jnp.zeros((N, H), _f32)
    q = _qinit(edge_attr, Wi)
    qagg = _scatter(q, dst3, zeros128)
    cnts = _counts(edge_index[1])
    cnt = cnts.sum(axis=0).reshape(-1)[:N].astype(_f32).reshape(N, 1)

    p, cinv = _prep(qagg, cnt, bit, b_init[None, :], W1b)

    be2 = b_e2[None, :]
    bu = b_u[None, :]
    for layer in range(NUM_LAYERS):
        g = _gather(p, src3)
        msg = _msg(edge_attr, g, W1ap, b1p, W_e2, be2)
        aggp = _scatter(msg, dst3, zeros128)
        if layer < NUM_LAYERS - 1:
            p = _update(aggp, cinv, W_u, bu, W1b, jnp.zeros((1, H), _f32))
        else:
            out = _update(aggp, cinv, W_u, bu, W_out, b_out[None, :])
    return out


# speedup vs baseline: 3.5936x; 1.1384x over previous
import functools

import jax
import jax.numpy as jnp
from jax import lax
from jax.experimental import pallas as pl
from jax.experimental.pallas import tpu as pltpu
from jax.experimental.pallas import tpu_sc as plsc

N = 10000
E = 320000
ED = 16
H = 128
OUT = 128
NUM_LAYERS = 3
EPS = 1e-5

NC = 2
NS = 16
NW = NC * NS
ROWS = E // 128
RPW = ROWS // NW
TAIL = ROWS - RPW * NW
STR = 624

_f32 = jnp.float32


def _mesh():
    return plsc.VectorSubcoreMesh(core_axis_name="c", subcore_axis_name="s")


def _striped_copy(s, src, dst):
    pltpu.sync_copy(src.at[pl.ds(s * STR, STR)], dst.at[pl.ds(s * STR, STR)])

    @pl.when(s == NS - 1)
    def _():
        rem = N - NS * STR
        pltpu.sync_copy(src.at[pl.ds(NS * STR, rem)], dst.at[pl.ds(NS * STR, rem)])


def _stats_body(ea_ref, sum_ref, sq_ref):
    i = pl.program_id(0)
    x = ea_ref[...]

    @pl.when(i == 0)
    def _():
        sum_ref[...] = jnp.zeros_like(sum_ref)
        sq_ref[...] = jnp.zeros_like(sq_ref)

    sum_ref[...] += jnp.sum(x, axis=0, keepdims=True)
    sq_ref[...] += jnp.sum(x * x, axis=0, keepdims=True)


def _stats(edge_attr):
    bs = 8000
    return pl.pallas_call(
        _stats_body,
        grid=(E // bs,),
        in_specs=[pl.BlockSpec((bs, ED), lambda i: (i, 0))],
        out_specs=[pl.BlockSpec((1, ED), lambda i: (0, 0))] * 2,
        out_shape=[jax.ShapeDtypeStruct((1, ED), _f32)] * 2,
    )(edge_attr)


KG = 3
NCH = RPW // KG
IPW = RPW * 128


def _gather_body(p_hbm, src_hbm, g_hbm, gb0, gb1, ibufall, gsem, wsem):
    c = lax.axis_index("c")
    s = lax.axis_index("s")
    w = c * NS + s
    base = w * RPW
    gbufs = (gb0, gb1)

    pltpu.sync_copy(src_hbm.at[pl.ds(w * IPW, IPW)], ibufall.at[pl.ds(0, IPW)])

    @pl.when(w < TAIL)
    def _():
        pltpu.sync_copy(src_hbm.at[pl.ds(NW * IPW + w * 128, 128)],
                        ibufall.at[pl.ds(IPW, 128)])

    def g_descs(k, b):
        return [
            pltpu.make_async_copy(
                p_hbm.at[ibufall.at[pl.ds((k * KG + j) * 128, 128)]],
                gbufs[b].at[pl.ds(j * 128, 128)], gsem)
            for j in range(KG)
        ]

    def wb_desc(k, b):
        r0 = base + k * KG
        return pltpu.make_async_copy(
            gbufs[b], g_hbm.at[pl.ds(r0 * 128, KG * 128)], wsem)

    @pl.loop(0, NCH // 2)
    def _o(o):
        for b in range(2):
            k = o * 2 + b

            @pl.when(o > 0)
            def _():
                wb_desc(k - 2, b).wait()

            descs = g_descs(k, b)
            for d in descs:
                d.start()
            for d in descs:
                d.wait()
            wb_desc(k, b).start()

    for b in range(2):
        wb_desc(NCH - 2 + b, b).wait()

    @pl.when(w < TAIL)
    def _tail():
        r = NW * RPW + w
        d = pltpu.make_async_copy(
            p_hbm.at[ibufall.at[pl.ds(IPW, 128)]],
            gbufs[0].at[pl.ds(0, 128)], gsem)
        d.start()
        d.wait()
        pltpu.sync_copy(gbufs[0].at[pl.ds(0, 128)], g_hbm.at[pl.ds(r * 128, 128)])


def _gather(p, src1d):
    fn = pl.kernel(
        _gather_body,
        out_type=jax.ShapeDtypeStruct((E, H), _f32),
        mesh=_mesh(),
        scratch_types=[
            pltpu.VMEM((KG * 128, H), _f32),
            pltpu.VMEM((KG * 128, H), _f32),
            pltpu.VMEM((IPW + 128,), jnp.int32),
            pltpu.SemaphoreType.DMA,
            pltpu.SemaphoreType.DMA,
        ],
    )
    return fn(p, src1d)


KC = 1


def _scatter_body(msg_hbm, dst3_hbm, zero128_hbm, agg_hbm,
                  mb0, mb1, ib0, ib1, acc, lsem, ssem):
    c = lax.axis_index("c")
    s = lax.axis_index("s")
    w = c * NS + s
    base = w * RPW
    mbufs = (mb0, mb1)
    ibufs = (ib0, ib1)
    nch = RPW

    _striped_copy(s, zero128_hbm, acc)
    plsc.subcore_barrier()

    def load_descs(k, b):
        r0 = base + k
        return [
            pltpu.make_async_copy(dst3_hbm.at[pl.ds(r0, 1)], ibufs[b], lsem),
            pltpu.make_async_copy(msg_hbm.at[pl.ds(r0 * 128, 128)], mbufs[b], lsem),
        ]

    def add_desc(k, b):
        return pltpu.make_async_copy(mbufs[b], acc.at[ibufs[b].at[0, 0]], ssem)

    for b in range(2):
        for d in load_descs(b, b):
            d.start()

    @pl.loop(0, nch // 2)
    def _o(o):
        for b in range(2):
            k = o * 2 + b
            for d in load_descs(k, b):
                d.wait()
            d = add_desc(k, b)
            d.start(add=True)
            d.wait()

            @pl.when(k + 2 < nch)
            def _():
                for d2 in load_descs(k + 2, b):
                    d2.start()

    @pl.when(w < TAIL)
    def _tail():
        r = NW * RPW + w
        pltpu.sync_copy(dst3_hbm.at[pl.ds(r, 1)], ib0)
        pltpu.sync_copy(msg_hbm.at[pl.ds(r * 128, 128)], mb0)
        pltpu.sync_copy(mb0, acc.at[ib0.at[0, 0]], add=True)

    plsc.subcore_barrier()
    _striped_copy(s, acc, agg_hbm.at[c])


def _scatter(msg, dst3, zeros128):
    fn = pl.kernel(
        _scatter_body,
        out_type=jax.ShapeDtypeStruct((NC, N, H), _f32),
        mesh=_mesh(),
        scratch_types=[
            pltpu.VMEM((128, H), _f32),
            pltpu.VMEM((128, H), _f32),
            pltpu.VMEM((1, 1, 128), jnp.int32),
            pltpu.VMEM((1, 1, 128), jnp.int32),
            pltpu.VMEM_SHARED((N, H), _f32),
            pltpu.SemaphoreType.DMA,
            pltpu.SemaphoreType.DMA,
        ],
    )
    return fn(msg, dst3, zeros128)


CR = 8
CW = 1280
HN = CR * CW // 2
IPW = RPW * 128


def _counts_body(dst_hbm, cnt_hbm, ibufall, cbuf, obuf):
    c = lax.axis_index("c")
    s = lax.axis_index("s")
    w = c * NS + s
    lane = lax.iota(jnp.int32, 16)
    ones = jnp.ones((16,), jnp.int32)
    zeros = jnp.zeros((16,), jnp.int32)

    pltpu.sync_copy(dst_hbm.at[pl.ds(w * IPW, IPW)], ibufall.at[pl.ds(0, IPW)])

    @pl.when(w < TAIL)
    def _():
        pltpu.sync_copy(dst_hbm.at[pl.ds(NW * IPW + w * 128, 128)],
                        ibufall.at[pl.ds(IPW, 128)])

    for half in range(2):
        lo = half * HN

        @pl.loop(0, HN, unroll=8)
        def _zero(i):
            cbuf[pl.ds(i * 16, 16)] = zeros

        def _count(i):
            idx = ibufall[pl.ds(i * 16, 16)]
            rel = idx - lo
            m = (rel >= 0) & (rel < HN)
            addr = rel * 16 + lane
            plsc.addupdate_scatter(cbuf, [addr], ones, mask=m)

        pl.loop(0, IPW // 16, unroll=4)(_count)

        @pl.when(w < TAIL)
        def _count_tail():
            pl.loop(IPW // 16, (IPW + 128) // 16)(_count)

        @pl.loop(0, HN // 16, unroll=2)
        def _reduce(gi):
            n0 = gi * 16
            base = n0 * 16 + lane * 16
            acc = plsc.load_gather(cbuf, [base])
            for l in range(1, 16):
                acc = acc + plsc.load_gather(cbuf, [base + l])
            flat = lo + n0
            obuf[flat // CW, pl.ds(flat % CW, 16)] = acc

    pltpu.sync_copy(obuf, cnt_hbm.at[w])


def _counts(dst1d):
    fn = pl.kernel(
        _counts_body,
        out_type=jax.ShapeDtypeStruct((NW, CR, CW), jnp.int32),
        mesh=_mesh(),
        scratch_types=[
            pltpu.VMEM((IPW + 128,), jnp.int32),
            pltpu.VMEM((HN * 16,), jnp.int32),
            pltpu.VMEM((CR, CW), jnp.int32),
        ],
        compiler_params=pltpu.CompilerParams(needs_layout_passes=False),
    )
    return fn(dst1d)


def _qinit_body(ea_ref, Wi_ref, q_ref):
    q_ref[...] = jnp.dot(ea_ref[...], Wi_ref[...], preferred_element_type=_f32)


def _qinit(edge_attr, Wi):
    return pl.pallas_call(
        _qinit_body,
        grid=(E // BM,),
        in_specs=[
            pl.BlockSpec((BM, ED), lambda i: (i, 0)),
            pl.BlockSpec((ED, H), lambda i: (0, 0)),
        ],
        out_specs=pl.BlockSpec((BM, H), lambda i: (i, 0)),
        out_shape=jax.ShapeDtypeStruct((E, H), _f32),
    )(edge_attr, Wi)


def _prep_body(qagg_ref, cnt_ref, bit_ref, bi_ref, W1b_ref, p_ref, cinv_ref):
    cnt = cnt_ref[...]
    rin = 1.0 / jnp.maximum(cnt, 1.0)
    nz = (cnt > 0.0).astype(_f32)
    x0 = jax.nn.relu(
        (qagg_ref[0] + qagg_ref[1]) * rin + nz * bit_ref[...] + bi_ref[...])
    p_ref[...] = jnp.dot(x0, W1b_ref[...], preferred_element_type=_f32)
    cinv_ref[...] = rin


def _prep(qagg, cnt, bit, bi, W1b):
    return pl.pallas_call(
        _prep_body,
        out_shape=[
            jax.ShapeDtypeStruct((N, H), _f32),
            jax.ShapeDtypeStruct((N, 1), _f32),
        ],
    )(qagg, cnt, bit, bi, W1b)


BM = 4000


def _msg_body(ea_ref, g_ref, W1_ref, b1_ref, W2_ref, b2_ref, out_ref):
    h = jax.nn.relu(
        jnp.dot(ea_ref[...], W1_ref[...], preferred_element_type=_f32)
        + g_ref[...] + b1_ref[...])
    out_ref[...] = jax.nn.relu(
        jnp.dot(h, W2_ref[...], preferred_element_type=_f32) + b2_ref[...])


def _msg(edge_attr, g, W1ap, b1p, We2, be2):
    return pl.pallas_call(
        _msg_body,
        grid=(E // BM,),
        in_specs=[
            pl.BlockSpec((BM, ED), lambda i: (i, 0)),
            pl.BlockSpec((BM, H), lambda i: (i, 0)),
            pl.BlockSpec((ED, H), lambda i: (0, 0)),
            pl.BlockSpec((1, H), lambda i: (0, 0)),
            pl.BlockSpec((H, H), lambda i: (0, 0)),
            pl.BlockSpec((1, H), lambda i: (0, 0)),
        ],
        out_specs=pl.BlockSpec((BM, H), lambda i: (i, 0)),
        out_shape=jax.ShapeDtypeStruct((E, H), _f32),
    )(edge_attr, g, W1ap, b1p, We2, be2)


def _update_body(aggp_ref, cinv_ref, Wu_ref, bu_ref, Wn_ref, bn_ref, out_ref):
    agg = (aggp_ref[0] + aggp_ref[1]) * cinv_ref[...]
    x = jax.nn.relu(
        jnp.dot(agg, Wu_ref[...], preferred_element_type=_f32) + bu_ref[...])
    out_ref[...] = jnp.dot(x, Wn_ref[...], preferred_element_type=_f32) + bn_ref[...]


def _update(aggp, cinv, Wu, bu, Wn, bn):
    return pl.pallas_call(
        _update_body,
        out_shape=jax.ShapeDtypeStruct((N, Wn.shape[1]), _f32),
    )(aggp, cinv, Wu, bu, Wn, bn)


def kernel(edge_index, edge_attr, bn_gamma, bn_beta, W_init, b_init,
           W_e1, b_e1, W_e2, b_e2, W_u, b_u, W_out, b_out):
    src1d = edge_index[0]
    dst3 = edge_index[1].reshape(ROWS, 1, 128)

    ssum, ssq = _stats(edge_attr)
    mu = ssum / float(E)
    var = ssq / float(E) - mu * mu
    sv = bn_gamma[None, :] * lax.rsqrt(var + EPS)
    tv = bn_beta[None, :] - mu * sv
    W1a = W_e1[:ED]
    W1b = W_e1[ED:]
    W1ap = W1a * sv.reshape(ED, 1)
    b1p = tv @ W1a + b_e1[None, :]
    Wi = W_init * sv.reshape(ED, 1)
    bit = tv @ W_init

    zeros128 = jnp.zeros((N, H), _f32)
    q = _qinit(edge_attr, Wi)
    qagg = _scatter(q, dst3, zeros128)
    cnts = _counts(edge_index[1])
    cnt = cnts.sum(axis=0).reshape(-1)[:N].astype(_f32).reshape(N, 1)

    p, cinv = _prep(qagg, cnt, bit, b_init[None, :], W1b)

    be2 = b_e2[None, :]
    bu = b_u[None, :]
    for layer in range(NUM_LAYERS):
        g = _gather(p, src1d)
        msg = _msg(edge_attr, g, W1ap, b1p, W_e2, be2)
        aggp = _scatter(msg, dst3, zeros128)
        if layer < NUM_LAYERS - 1:
            p = _update(aggp, cinv, W_u, bu, W1b, jnp.zeros((1, H), _f32))
        else:
            out = _update(aggp, cinv, W_u, bu, W_out, b_out[None, :])
    return out


# speedup vs baseline: 4.4254x; 1.2315x over previous
import functools

import jax
import jax.numpy as jnp
from jax import lax
from jax.experimental import pallas as pl
from jax.experimental.pallas import tpu as pltpu
from jax.experimental.pallas import tpu_sc as plsc

N = 10000
E = 320000
ED = 16
H = 128
OUT = 128
NUM_LAYERS = 3
EPS = 1e-5

NC = 2
NS = 16
NW = NC * NS
ROWS = E // 128
RPW = ROWS // NW
TAIL = ROWS - RPW * NW
STR = 624

_f32 = jnp.float32


def _mesh():
    return plsc.VectorSubcoreMesh(core_axis_name="c", subcore_axis_name="s")


def _striped_copy(s, src, dst):
    pltpu.sync_copy(src.at[pl.ds(s * STR, STR)], dst.at[pl.ds(s * STR, STR)])

    @pl.when(s == NS - 1)
    def _():
        rem = N - NS * STR
        pltpu.sync_copy(src.at[pl.ds(NS * STR, rem)], dst.at[pl.ds(NS * STR, rem)])


def _stats_body(ea_ref, sum_ref, sq_ref):
    i = pl.program_id(0)
    x = ea_ref[...]

    @pl.when(i == 0)
    def _():
        sum_ref[...] = jnp.zeros_like(sum_ref)
        sq_ref[...] = jnp.zeros_like(sq_ref)

    sum_ref[...] += jnp.sum(x, axis=1, keepdims=True)
    sq_ref[...] += jnp.sum(x * x, axis=1, keepdims=True)


def _stats(eaT):
    bs = 16000
    return pl.pallas_call(
        _stats_body,
        grid=(E // bs,),
        in_specs=[pl.BlockSpec((ED, bs), lambda i: (0, i))],
        out_specs=[pl.BlockSpec((ED, 1), lambda i: (0, 0))] * 2,
        out_shape=[jax.ShapeDtypeStruct((ED, 1), _f32)] * 2,
    )(eaT)


KG = 3
NCH = RPW // KG
IPW = RPW * 128


def _gather_body(p_hbm, src_hbm, g_hbm, gb0, gb1, ibufall, gsem, wsem):
    c = lax.axis_index("c")
    s = lax.axis_index("s")
    w = c * NS + s
    base = w * RPW
    gbufs = (gb0, gb1)

    pltpu.sync_copy(src_hbm.at[pl.ds(w * IPW, IPW)], ibufall.at[pl.ds(0, IPW)])

    @pl.when(w < TAIL)
    def _():
        pltpu.sync_copy(src_hbm.at[pl.ds(NW * IPW + w * 128, 128)],
                        ibufall.at[pl.ds(IPW, 128)])

    def g_descs(k, b):
        return [
            pltpu.make_async_copy(
                p_hbm.at[ibufall.at[pl.ds((k * KG + j) * 128, 128)]],
                gbufs[b].at[pl.ds(j * 128, 128)], gsem)
            for j in range(KG)
        ]

    def wb_desc(k, b):
        r0 = base + k * KG
        return pltpu.make_async_copy(
            gbufs[b], g_hbm.at[pl.ds(r0 * 128, KG * 128)], wsem)

    @pl.loop(0, NCH // 2)
    def _o(o):
        for b in range(2):
            k = o * 2 + b

            @pl.when(o > 0)
            def _():
                wb_desc(k - 2, b).wait()

            descs = g_descs(k, b)
            for d in descs:
                d.start()
            for d in descs:
                d.wait()
            wb_desc(k, b).start()

    for b in range(2):
        wb_desc(NCH - 2 + b, b).wait()

    @pl.when(w < TAIL)
    def _tail():
        r = NW * RPW + w
        d = pltpu.make_async_copy(
            p_hbm.at[ibufall.at[pl.ds(IPW, 128)]],
            gbufs[0].at[pl.ds(0, 128)], gsem)
        d.start()
        d.wait()
        pltpu.sync_copy(gbufs[0].at[pl.ds(0, 128)], g_hbm.at[pl.ds(r * 128, 128)])


def _gather(p, src1d):
    fn = pl.kernel(
        _gather_body,
        out_type=jax.ShapeDtypeStruct((E, H), _f32),
        mesh=_mesh(),
        scratch_types=[
            pltpu.VMEM((KG * 128, H), _f32),
            pltpu.VMEM((KG * 128, H), _f32),
            pltpu.VMEM((IPW + 128,), jnp.int32),
            pltpu.SemaphoreType.DMA,
            pltpu.SemaphoreType.DMA,
        ],
    )
    return fn(p, src1d)


KC = 1


def _scatter_body(msg_hbm, dst3_hbm, zero128_hbm, agg_hbm,
                  mb0, mb1, ib0, ib1, acc, lsem, ssem):
    c = lax.axis_index("c")
    s = lax.axis_index("s")
    w = c * NS + s
    base = w * RPW
    mbufs = (mb0, mb1)
    ibufs = (ib0, ib1)
    nch = RPW

    _striped_copy(s, zero128_hbm, acc)
    plsc.subcore_barrier()

    def load_descs(k, b):
        r0 = base + k
        return [
            pltpu.make_async_copy(dst3_hbm.at[pl.ds(r0, 1)], ibufs[b], lsem),
            pltpu.make_async_copy(msg_hbm.at[pl.ds(r0 * 128, 128)], mbufs[b], lsem),
        ]

    def add_desc(k, b):
        return pltpu.make_async_copy(mbufs[b], acc.at[ibufs[b].at[0, 0]], ssem)

    for b in range(2):
        for d in load_descs(b, b):
            d.start()

    @pl.loop(0, nch // 2)
    def _o(o):
        for b in range(2):
            k = o * 2 + b
            for d in load_descs(k, b):
                d.wait()
            d = add_desc(k, b)
            d.start(add=True)
            d.wait()

            @pl.when(k + 2 < nch)
            def _():
                for d2 in load_descs(k + 2, b):
                    d2.start()

    @pl.when(w < TAIL)
    def _tail():
        r = NW * RPW + w
        pltpu.sync_copy(dst3_hbm.at[pl.ds(r, 1)], ib0)
        pltpu.sync_copy(msg_hbm.at[pl.ds(r * 128, 128)], mb0)
        pltpu.sync_copy(mb0, acc.at[ib0.at[0, 0]], add=True)

    plsc.subcore_barrier()
    _striped_copy(s, acc, agg_hbm.at[c])


def _scatter(msg, dst3, zeros128):
    fn = pl.kernel(
        _scatter_body,
        out_type=jax.ShapeDtypeStruct((NC, N, H), _f32),
        mesh=_mesh(),
        scratch_types=[
            pltpu.VMEM((128, H), _f32),
            pltpu.VMEM((128, H), _f32),
            pltpu.VMEM((1, 1, 128), jnp.int32),
            pltpu.VMEM((1, 1, 128), jnp.int32),
            pltpu.VMEM_SHARED((N, H), _f32),
            pltpu.SemaphoreType.DMA,
            pltpu.SemaphoreType.DMA,
        ],
    )
    return fn(msg, dst3, zeros128)


CR = 8
CW = 1280
HN = CR * CW // 2
IPW = RPW * 128


def _counts_body(dst_hbm, cnt_hbm, ibufall, cbuf, obuf):
    c = lax.axis_index("c")
    s = lax.axis_index("s")
    w = c * NS + s
    lane = lax.iota(jnp.int32, 16)
    ones = jnp.ones((16,), jnp.int32)
    zeros = jnp.zeros((16,), jnp.int32)

    pltpu.sync_copy(dst_hbm.at[pl.ds(w * IPW, IPW)], ibufall.at[pl.ds(0, IPW)])

    @pl.when(w < TAIL)
    def _():
        pltpu.sync_copy(dst_hbm.at[pl.ds(NW * IPW + w * 128, 128)],
                        ibufall.at[pl.ds(IPW, 128)])

    for half in range(2):
        lo = half * HN

        @pl.loop(0, HN, unroll=8)
        def _zero(i):
            cbuf[pl.ds(i * 16, 16)] = zeros

        def _count(i):
            idx = ibufall[pl.ds(i * 16, 16)]
            rel = idx - lo
            m = (rel >= 0) & (rel < HN)
            addr = rel * 16 + lane
            plsc.addupdate_scatter(cbuf, [addr], ones, mask=m)

        pl.loop(0, IPW // 16, unroll=4)(_count)

        @pl.when(w < TAIL)
        def _count_tail():
            pl.loop(IPW // 16, (IPW + 128) // 16)(_count)

        @pl.loop(0, HN // 16, unroll=2)
        def _reduce(gi):
            n0 = gi * 16
            base = n0 * 16 + lane * 16
            acc = plsc.load_gather(cbuf, [base])
            for l in range(1, 16):
                acc = acc + plsc.load_gather(cbuf, [base + l])
            flat = lo + n0
            obuf[flat // CW, pl.ds(flat % CW, 16)] = acc

    pltpu.sync_copy(obuf, cnt_hbm.at[w])


def _counts(dst1d):
    fn = pl.kernel(
        _counts_body,
        out_type=jax.ShapeDtypeStruct((NW, CR, CW), jnp.int32),
        mesh=_mesh(),
        scratch_types=[
            pltpu.VMEM((IPW + 128,), jnp.int32),
            pltpu.VMEM((HN * 16,), jnp.int32),
            pltpu.VMEM((CR, CW), jnp.int32),
        ],
        compiler_params=pltpu.CompilerParams(needs_layout_passes=False),
    )
    return fn(dst1d)


def _dotT(a, b):
    return lax.dot_general(a, b, ((((0,), (0,))), ((), ())),
                           preferred_element_type=_f32)


def _qinit_body(ea_ref, Wi_ref, q_ref):
    q_ref[...] = _dotT(ea_ref[...], Wi_ref[...])


def _qinit(eaT, Wi):
    return pl.pallas_call(
        _qinit_body,
        grid=(E // BM,),
        in_specs=[
            pl.BlockSpec((ED, BM), lambda i: (0, i)),
            pl.BlockSpec((ED, H), lambda i: (0, 0)),
        ],
        out_specs=pl.BlockSpec((BM, H), lambda i: (i, 0)),
        out_shape=jax.ShapeDtypeStruct((E, H), _f32),
    )(eaT, Wi)


def _prep_body(qagg_ref, cnt_ref, bit_ref, bi_ref, W1b_ref, p_ref, cinv_ref):
    cnt = cnt_ref[...]
    rin = 1.0 / jnp.maximum(cnt, 1.0)
    nz = (cnt > 0.0).astype(_f32)
    x0 = jax.nn.relu(
        (qagg_ref[0] + qagg_ref[1]) * rin + nz * bit_ref[...] + bi_ref[...])
    p_ref[...] = jnp.dot(x0, W1b_ref[...], preferred_element_type=_f32)
    cinv_ref[...] = rin


def _prep(qagg, cnt, bit, bi, W1b):
    return pl.pallas_call(
        _prep_body,
        out_shape=[
            jax.ShapeDtypeStruct((N, H), _f32),
            jax.ShapeDtypeStruct((N, 1), _f32),
        ],
    )(qagg, cnt, bit, bi, W1b)


BM = 6400


def _msg_body(ea_ref, g_ref, W1_ref, b1_ref, W2_ref, b2_ref, out_ref):
    h = jax.nn.relu(_dotT(ea_ref[...], W1_ref[...]) + g_ref[...] + b1_ref[...])
    out_ref[...] = jax.nn.relu(
        jnp.dot(h, W2_ref[...], preferred_element_type=_f32) + b2_ref[...])


def _msg(eaT, g, W1ap, b1p, We2, be2):
    return pl.pallas_call(
        _msg_body,
        grid=(E // BM,),
        in_specs=[
            pl.BlockSpec((ED, BM), lambda i: (0, i)),
            pl.BlockSpec((BM, H), lambda i: (i, 0)),
            pl.BlockSpec((ED, H), lambda i: (0, 0)),
            pl.BlockSpec((1, H), lambda i: (0, 0)),
            pl.BlockSpec((H, H), lambda i: (0, 0)),
            pl.BlockSpec((1, H), lambda i: (0, 0)),
        ],
        out_specs=pl.BlockSpec((BM, H), lambda i: (i, 0)),
        out_shape=jax.ShapeDtypeStruct((E, H), _f32),
    )(eaT, g, W1ap, b1p, We2, be2)


def _update_body(aggp_ref, cinv_ref, Wu_ref, bu_ref, Wn_ref, bn_ref, out_ref):
    agg = (aggp_ref[0] + aggp_ref[1]) * cinv_ref[...]
    x = jax.nn.relu(
        jnp.dot(agg, Wu_ref[...], preferred_element_type=_f32) + bu_ref[...])
    out_ref[...] = jnp.dot(x, Wn_ref[...], preferred_element_type=_f32) + bn_ref[...]


def _update(aggp, cinv, Wu, bu, Wn, bn):
    return pl.pallas_call(
        _update_body,
        out_shape=jax.ShapeDtypeStruct((N, Wn.shape[1]), _f32),
    )(aggp, cinv, Wu, bu, Wn, bn)


def kernel(edge_index, edge_attr, bn_gamma, bn_beta, W_init, b_init,
           W_e1, b_e1, W_e2, b_e2, W_u, b_u, W_out, b_out):
    src1d = edge_index[0]
    dst3 = edge_index[1].reshape(ROWS, 1, 128)

    eaT = edge_attr.T
    ssum, ssq = _stats(eaT)
    mu = ssum.reshape(1, ED) / float(E)
    var = ssq.reshape(1, ED) / float(E) - mu * mu
    sv = bn_gamma[None, :] * lax.rsqrt(var + EPS)
    tv = bn_beta[None, :] - mu * sv
    W1a = W_e1[:ED]
    W1b = W_e1[ED:]
    W1ap = W1a * sv.reshape(ED, 1)
    b1p = tv @ W1a + b_e1[None, :]
    Wi = W_init * sv.reshape(ED, 1)
    bit = tv @ W_init

    zeros128 = jnp.zeros((N, H), _f32)
    q = _qinit(eaT, Wi)
    qagg = _scatter(q, dst3, zeros128)
    cnts = _counts(edge_index[1])
    cnt = cnts.sum(axis=0).reshape(-1)[:N].astype(_f32).reshape(N, 1)

    p, cinv = _prep(qagg, cnt, bit, b_init[None, :], W1b)

    be2 = b_e2[None, :]
    bu = b_u[None, :]
    for layer in range(NUM_LAYERS):
        g = _gather(p, src1d)
        msg = _msg(eaT, g, W1ap, b1p, W_e2, be2)
        aggp = _scatter(msg, dst3, zeros128)
        if layer < NUM_LAYERS - 1:
            p = _update(aggp, cinv, W_u, bu, W1b, jnp.zeros((1, H), _f32))
        else:
            out = _update(aggp, cinv, W_u, bu, W_out, b_out[None, :])
    return out
